# baseline (speedup 1.0000x reference)
import functools
import jax
import jax.numpy as jnp
from jax import lax
from jax.experimental import pallas as pl
from jax.experimental.pallas import tpu as pltpu
from jax.experimental.pallas import tpu_sc as plsc

N = 100000
E = 50
L = 16
NC = 2
NS = 16
NW = NC * NS
CH = 512
NCHUNKS = (N + CH - 1) // CH
TAIL_ROWS = N - (NCHUNKS - 1) * CH
ITERS = (NCHUNKS + NW - 1) // NW


def _body(x_hbm, ae_hbm, out_hbm, xbuf, aebuf, outbuf):
    wid = lax.axis_index("s") * NC + lax.axis_index("c")
    lanes = lax.iota(jnp.int32, L)

    pltpu.sync_copy(ae_hbm, aebuf)
    ae_vecs = [aebuf[pl.ds(L * k, L)] for k in range(E)]

    def compute(groups):
        @pl.loop(0, groups)
        def _(g):
            idx0 = (g * L + lanes) * E
            acc = plsc.load_gather(xbuf, [idx0]) * ae_vecs[0]
            for k in range(1, E):
                acc = acc + plsc.load_gather(xbuf, [idx0 + k]) * ae_vecs[k]
            outbuf[pl.ds(g * L, L)] = acc

    for i in range(ITERS):
        c = wid + i * NW

        @pl.when(c < NCHUNKS - 1)
        def _():
            base = pl.multiple_of(c * (CH * E), 8)
            pltpu.sync_copy(x_hbm.at[pl.ds(base, CH * E)], xbuf)
            compute(CH // L)
            pltpu.sync_copy(outbuf, out_hbm.at[pl.ds(c * CH, CH)])

        @pl.when(c == NCHUNKS - 1)
        def _():
            base = pl.multiple_of(c * (CH * E), 8)
            nw = TAIL_ROWS * E
            pltpu.sync_copy(x_hbm.at[pl.ds(base, nw)], xbuf.at[pl.ds(0, nw)])
            compute(TAIL_ROWS // L)
            pltpu.sync_copy(outbuf.at[pl.ds(0, TAIL_ROWS)],
                            out_hbm.at[pl.ds(c * CH, TAIL_ROWS)])


@functools.partial(
    pl.kernel,
    out_type=jax.ShapeDtypeStruct((N,), jnp.float32),
    mesh=plsc.VectorSubcoreMesh(core_axis_name="c", subcore_axis_name="s"),
    compiler_params=pltpu.CompilerParams(needs_layout_passes=False),
    scratch_types=[
        pltpu.VMEM((CH * E,), jnp.float32),
        pltpu.VMEM((E * L,), jnp.float32),
        pltpu.VMEM((CH,), jnp.float32),
    ],
)
def _sc_matvec(x_flat, ae_exp, out_flat, xbuf, aebuf, outbuf):
    _body(x_flat, ae_exp, out_flat, xbuf, aebuf, outbuf)


@jax.jit
def kernel(x, atomic_energies):
    ae_exp = jnp.broadcast_to(atomic_energies[:, None], (E, L)).reshape(E * L)
    out = _sc_matvec(x.reshape(N * E), ae_exp)
    return out[:, None]
